# Initial kernel scaffold; baseline (speedup 1.0000x reference)
import jax
import jax.numpy as jnp
from jax.experimental import pallas as pl


def kernel(nodes_2x2, ancS, ancL, W1, b1, W2, b2):
    raise NotImplementedError("write your pallas kernel here")


# speedup vs baseline: 5.3791x; 5.3791x over previous
import functools

import jax
import jax.numpy as jnp
from jax import lax
from jax.experimental import pallas as pl
from jax.experimental.pallas import tpu as pltpu
from jax.experimental.pallas import tpu_sc as plsc

_B = 4096
_M = 4096
_D = 256
_K = 16
_TAU = 0.3

_NC = 2
_NS = 16
_NW = _NC * _NS
_ROWS = _B // _NW
_GROUPS = _M // 32

_INF = float("inf")


def _merge16(bk, bv, ck, cv):
    sk, sv = plsc.sort_key_val(ck, cv)
    rk = lax.rev(sk, (0,))
    rv = lax.rev(sv, (0,))
    keep = bk <= rk
    nk = jnp.minimum(bk, rk)
    nv = jnp.where(keep, bv, rv)
    return plsc.sort_key_val(nk, nv)


def _sc_topk(qx_s, qy_s, ax_s, ay_s, qx_l, qy_l, ax_l, ay_l):
    mesh = plsc.VectorSubcoreMesh(
        core_axis_name="c", subcore_axis_name="s",
        num_cores=_NC, num_subcores=_NS)

    f32 = jnp.float32
    out_sds = jax.ShapeDtypeStruct((_B, _K), f32)

    @functools.partial(
        pl.kernel, mesh=mesh,
        out_type=(out_sds,) * 6,
        compiler_params=pltpu.CompilerParams(needs_layout_passes=False),
        scratch_types=[
            pltpu.VMEM((_ROWS, _K), f32),
            pltpu.VMEM((_ROWS, _K), f32),
            pltpu.VMEM((_M,), f32),
            pltpu.VMEM((_M,), f32),
            pltpu.VMEM((_M,), f32),
            pltpu.VMEM((_M,), f32),
            pltpu.VMEM((_ROWS, _K), f32),
            pltpu.VMEM((_ROWS, _K), f32),
            pltpu.VMEM((_ROWS, _K), f32),
            pltpu.SemaphoreType.DMA,
            pltpu.SemaphoreType.DMA,
            pltpu.SemaphoreType.DMA,
            pltpu.SemaphoreType.DMA,
        ],
    )
    def k(qxs_h, qys_h, axs_h, ays_h, qxl_h, qyl_h, axl_h, ayl_h,
          oxs_h, oys_h, ows_h, oxl_h, oyl_h, owl_h,
          qxb, qyb, bufx0, bufx1, bufy0, bufy1, oxb, oyb, owb,
          semx0, semx1, semy0, semy1):
        wid = lax.axis_index("s") * _NC + lax.axis_index("c")
        base = wid * _ROWS

        iota = lax.broadcasted_iota(jnp.int32, (_K,), 0)
        bufs = ((bufx0, bufy0, semx0, semy0), (bufx1, bufy1, semx1, semy1))

        chans = (
            (qxs_h, qys_h, axs_h, ays_h, oxs_h, oys_h, ows_h),
            (qxl_h, qyl_h, axl_h, ayl_h, oxl_h, oyl_h, owl_h),
        )
        for (qx_h, qy_h, ax_h, ay_h, ox_h, oy_h, ow_h) in chans:
            pltpu.sync_copy(qx_h.at[pl.ds(base, _ROWS)], qxb)
            pltpu.sync_copy(qy_h.at[pl.ds(base, _ROWS)], qyb)
            pltpu.async_copy(ax_h.at[base], bufx0, semx0)
            pltpu.async_copy(ay_h.at[base], bufy0, semy0)

            def compute_row(r, arx, ary):
                qx = qxb[r]
                qy = qyb[r]

                def group_body(g, carry):
                    bk, bv, wth = carry
                    o = g * 32

                    def half(off):
                        dx = arx[pl.ds(off, _K)] - qx
                        dy = ary[pl.ds(off, _K)] - qy
                        return dx * dx + dy * dy

                    c0 = half(o)
                    c1 = half(o + _K)
                    mn = jnp.min(jnp.minimum(c0, c1))

                    def do_merge(args):
                        bk, bv = args
                        bk, bv = _merge16(bk, bv, c0, o + iota)
                        bk, bv = _merge16(bk, bv, c1, o + _K + iota)
                        return bk, bv, jnp.max(bk)

                    def no_merge(args):
                        bk, bv = args
                        return bk, bv, wth

                    return lax.cond(mn < wth, do_merge, no_merge, (bk, bv))

                bk0 = jnp.full((_K,), _INF, f32)
                bv0 = jnp.zeros((_K,), jnp.int32)
                bk, bv, wmax = lax.fori_loop(
                    0, _GROUPS, group_body, (bk0, bv0, jnp.float32(_INF)))

                e = jnp.exp((bk - wmax) * (1.0 / _TAU))
                oxb[r] = plsc.load_gather(arx, [bv])
                oyb[r] = plsc.load_gather(ary, [bv])
                owb[r] = e

            def row_pair(rr, _, ax_h=ax_h, ay_h=ay_h):
                for par in range(2):
                    r = 2 * rr + par
                    arx, ary, sx, sy = bufs[par]
                    nbufx, nbufy, nsx, nsy = bufs[1 - par]

                    @pl.when(r + 1 < _ROWS)
                    def _():
                        pltpu.async_copy(ax_h.at[base + r + 1], nbufx, nsx)
                        pltpu.async_copy(ay_h.at[base + r + 1], nbufy, nsy)

                    pltpu.make_async_copy(ax_h.at[base + r], arx, sx).wait()
                    pltpu.make_async_copy(ay_h.at[base + r], ary, sy).wait()
                    compute_row(r, arx, ary)
                return 0

            lax.fori_loop(0, _ROWS // 2, row_pair, 0)
            pltpu.sync_copy(oxb, ox_h.at[pl.ds(base, _ROWS)])
            pltpu.sync_copy(oyb, oy_h.at[pl.ds(base, _ROWS)])
            pltpu.sync_copy(owb, ow_h.at[pl.ds(base, _ROWS)])

    return k(qx_s, qy_s, ax_s, ay_s, qx_l, qy_l, ax_l, ay_l)


def _gelu(x):
    return 0.5 * x * (1.0 + lax.erf(x * (1.0 / jnp.sqrt(2.0).astype(x.dtype))))


_CH = 8192


def _mlp_body(x_ref, w1_ref, b1_ref, w2_ref, b2_ref, o_ref, esum_ref):
    i = pl.program_id(0)
    ni = pl.num_programs(0)
    x = x_ref[...]
    a = x[:, 0:2]
    wgt = x[:, 2:3]
    h1 = _gelu(jnp.dot(a, w1_ref[...], preferred_element_type=jnp.float32)
               + b1_ref[...])
    h2 = _gelu(jnp.dot(h1, w2_ref[...], preferred_element_type=jnp.float32)
               + b2_ref[...])
    h2 = h2 * wgt

    @pl.when(i == 0)
    def _():
        o_ref[...] = jnp.zeros_like(o_ref)
        esum_ref[...] = jnp.zeros_like(esum_ref)

    o_ref[...] += h2[0:_B, :] + h2[_B:_CH, :]
    esum_ref[...] += wgt[0:_B, :] + wgt[_B:_CH, :]

    @pl.when(i == ni - 1)
    def _():
        o_ref[...] = o_ref[...] / esum_ref[...]


def _tc_mlp(x, w1t, b1, w2t, b2):
    grid = (_K * _B) // _CH
    return pl.pallas_call(
        _mlp_body,
        grid=(grid,),
        in_specs=[
            pl.BlockSpec((_CH, 4), lambda i: (i, 0)),
            pl.BlockSpec((2, _D), lambda i: (0, 0)),
            pl.BlockSpec((1, _D), lambda i: (0, 0)),
            pl.BlockSpec((_D, _D), lambda i: (0, 0)),
            pl.BlockSpec((1, _D), lambda i: (0, 0)),
        ],
        out_specs=pl.BlockSpec((_B, _D), lambda i: (0, 0)),
        out_shape=jax.ShapeDtypeStruct((_B, _D), jnp.float32),
        scratch_shapes=[pltpu.VMEM((_B, 1), jnp.float32)],
    )(x, w1t, b1, w2t, b2)


def kernel(nodes_2x2, ancS, ancL, W1, b1, W2, b2):
    gs = nodes_2x2[:, 0, :]
    gl = nodes_2x2[:, 1, :]
    qx_s = jnp.broadcast_to(gs[:, 0:1], (_B, _K))
    qy_s = jnp.broadcast_to(gs[:, 1:2], (_B, _K))
    qx_l = jnp.broadcast_to(gl[:, 0:1], (_B, _K))
    qy_l = jnp.broadcast_to(gl[:, 1:2], (_B, _K))
    ax_s = ancS[:, :, 0]
    ay_s = ancS[:, :, 1]
    ax_l = ancL[:, :, 0]
    ay_l = ancL[:, :, 1]

    oxs, oys, ows, oxl, oyl, owl = _sc_topk(
        qx_s, qy_s, ax_s, ay_s, qx_l, qy_l, ax_l, ay_l)

    w1t = W1.T
    w2t = W2.T
    b1r = b1.reshape(1, _D)
    b2r = b2.reshape(1, _D)

    def assemble(ox, oy, ow):
        cols = [ox.T.reshape(-1), oy.T.reshape(-1), ow.T.reshape(-1),
                jnp.zeros((_K * _B,), jnp.float32)]
        return jnp.stack(cols, axis=-1)

    hs = _tc_mlp(assemble(oxs, oys, ows), w1t, b1r, w2t, b2r)
    hl = _tc_mlp(assemble(oxl, oyl, owl), w1t, b1r, w2t, b2r)
    return (hs, hl)


# speedup vs baseline: 7.0537x; 1.3113x over previous
import functools

import jax
import jax.numpy as jnp
from jax import lax
from jax.experimental import pallas as pl
from jax.experimental.pallas import tpu as pltpu
from jax.experimental.pallas import tpu_sc as plsc

_B = 4096
_M = 4096
_D = 256
_K = 16
_TAU = 0.3

_NC = 2
_NS = 16
_NW = _NC * _NS
_ROWS = _B // _NW
_GROUPS = _M // 32

_INF = float("inf")


def _merge16(bk, bv, ck, cv):
    sk, sv = plsc.sort_key_val(ck, cv)
    rk = lax.rev(sk, (0,))
    rv = lax.rev(sv, (0,))
    keep = bk <= rk
    nk = jnp.minimum(bk, rk)
    nv = jnp.where(keep, bv, rv)
    return plsc.sort_key_val(nk, nv)


def _sc_topk(qx_s, qy_s, anc_s, qx_l, qy_l, anc_l):
    mesh = plsc.VectorSubcoreMesh(
        core_axis_name="c", subcore_axis_name="s",
        num_cores=_NC, num_subcores=_NS)

    f32 = jnp.float32
    out_sds = jax.ShapeDtypeStruct((_B, _K), f32)

    @functools.partial(
        pl.kernel, mesh=mesh,
        out_type=(out_sds,) * 6,
        compiler_params=pltpu.CompilerParams(needs_layout_passes=False),
        scratch_types=[
            pltpu.VMEM((_ROWS, _K), f32),
            pltpu.VMEM((_ROWS, _K), f32),
            pltpu.VMEM((2, _M), f32),
            pltpu.VMEM((2, _M), f32),
            pltpu.VMEM((_ROWS, _K), f32),
            pltpu.VMEM((_ROWS, _K), f32),
            pltpu.VMEM((_ROWS, _K), f32),
            pltpu.SemaphoreType.DMA,
            pltpu.SemaphoreType.DMA,
        ],
    )
    def k(qxs_h, qys_h, ancs_h, qxl_h, qyl_h, ancl_h,
          oxs_h, oys_h, ows_h, oxl_h, oyl_h, owl_h,
          qxb, qyb, buf0, buf1, oxb, oyb, owb, sem0, sem1):
        wid = lax.axis_index("s") * _NC + lax.axis_index("c")
        base = wid * _ROWS

        iota = lax.broadcasted_iota(jnp.int32, (_K,), 0)
        zero16 = jnp.zeros((_K,), jnp.int32)
        one16 = jnp.full((_K,), 1, jnp.int32)
        bufs = ((buf0, sem0), (buf1, sem1))

        chans = (
            (qxs_h, qys_h, ancs_h, oxs_h, oys_h, ows_h),
            (qxl_h, qyl_h, ancl_h, oxl_h, oyl_h, owl_h),
        )
        for (qx_h, qy_h, anc_h, ox_h, oy_h, ow_h) in chans:
            pltpu.sync_copy(qx_h.at[pl.ds(base, _ROWS)], qxb)
            pltpu.sync_copy(qy_h.at[pl.ds(base, _ROWS)], qyb)
            pltpu.async_copy(anc_h.at[base], buf0, sem0)

            def compute_row(r, buf):
                qx = qxb[r]
                qy = qyb[r]

                def group_body(g, carry):
                    bk, bv, wth = carry
                    o = g * 32

                    def half(off):
                        dx = buf[0, pl.ds(off, _K)] - qx
                        dy = buf[1, pl.ds(off, _K)] - qy
                        return dx * dx + dy * dy

                    c0 = half(o)
                    c1 = half(o + _K)
                    hits = plsc.all_reduce_population_count(
                        jnp.minimum(c0, c1) < wth)[0]

                    def do_merge(args):
                        bk, bv = args
                        bk, bv = _merge16(bk, bv, c0, o + iota)
                        bk, bv = _merge16(bk, bv, c1, o + _K + iota)
                        return bk, bv, bk[_K - 1]

                    def no_merge(args):
                        bk, bv = args
                        return bk, bv, wth

                    return lax.cond(hits > 0, do_merge, no_merge, (bk, bv))

                bk0 = jnp.full((_K,), _INF, f32)
                bv0 = jnp.zeros((_K,), jnp.int32)
                bk, bv, wmax = lax.fori_loop(
                    0, _GROUPS, group_body, (bk0, bv0, jnp.float32(_INF)),
                    unroll=2)

                e = jnp.exp((bk - wmax) * (1.0 / _TAU))
                oxb[r] = plsc.load_gather(buf, [zero16, bv])
                oyb[r] = plsc.load_gather(buf, [one16, bv])
                owb[r] = e

            def row_pair(rr, _, anc_h=anc_h):
                for par in range(2):
                    r = 2 * rr + par
                    buf, sem = bufs[par]
                    nbuf, nsem = bufs[1 - par]

                    @pl.when(r + 1 < _ROWS)
                    def _():
                        pltpu.async_copy(anc_h.at[base + r + 1], nbuf, nsem)

                    pltpu.make_async_copy(anc_h.at[base + r], buf, sem).wait()
                    compute_row(r, buf)
                return 0

            lax.fori_loop(0, _ROWS // 2, row_pair, 0)
            pltpu.sync_copy(oxb, ox_h.at[pl.ds(base, _ROWS)])
            pltpu.sync_copy(oyb, oy_h.at[pl.ds(base, _ROWS)])
            pltpu.sync_copy(owb, ow_h.at[pl.ds(base, _ROWS)])

    return k(qx_s, qy_s, anc_s, qx_l, qy_l, anc_l)


def _gelu(x):
    return 0.5 * x * (1.0 + lax.erf(x * (1.0 / jnp.sqrt(2.0).astype(x.dtype))))


_CH = 8192


def _mlp_body(x_ref, w1_ref, b1_ref, w2_ref, b2_ref, o_ref, esum_ref):
    i = pl.program_id(0)
    ni = pl.num_programs(0)
    x = x_ref[...]
    a = x[:, 0:2]
    wgt = x[:, 2:3]
    h1 = _gelu(jnp.dot(a, w1_ref[...], preferred_element_type=jnp.float32)
               + b1_ref[...])
    h2 = _gelu(jnp.dot(h1, w2_ref[...], preferred_element_type=jnp.float32)
               + b2_ref[...])
    h2 = h2 * wgt

    @pl.when(i == 0)
    def _():
        o_ref[...] = jnp.zeros_like(o_ref)
        esum_ref[...] = jnp.zeros_like(esum_ref)

    o_ref[...] += h2[0:_B, :] + h2[_B:_CH, :]
    esum_ref[...] += wgt[0:_B, :] + wgt[_B:_CH, :]

    @pl.when(i == ni - 1)
    def _():
        o_ref[...] = o_ref[...] / esum_ref[...]


def _tc_mlp(x, w1t, b1, w2t, b2):
    grid = (_K * _B) // _CH
    return pl.pallas_call(
        _mlp_body,
        grid=(grid,),
        in_specs=[
            pl.BlockSpec((_CH, 4), lambda i: (i, 0)),
            pl.BlockSpec((2, _D), lambda i: (0, 0)),
            pl.BlockSpec((1, _D), lambda i: (0, 0)),
            pl.BlockSpec((_D, _D), lambda i: (0, 0)),
            pl.BlockSpec((1, _D), lambda i: (0, 0)),
        ],
        out_specs=pl.BlockSpec((_B, _D), lambda i: (0, 0)),
        out_shape=jax.ShapeDtypeStruct((_B, _D), jnp.float32),
        scratch_shapes=[pltpu.VMEM((_B, 1), jnp.float32)],
    )(x, w1t, b1, w2t, b2)


def kernel(nodes_2x2, ancS, ancL, W1, b1, W2, b2):
    gs = nodes_2x2[:, 0, :]
    gl = nodes_2x2[:, 1, :]
    qx_s = jnp.broadcast_to(gs[:, 0:1], (_B, _K))
    qy_s = jnp.broadcast_to(gs[:, 1:2], (_B, _K))
    qx_l = jnp.broadcast_to(gl[:, 0:1], (_B, _K))
    qy_l = jnp.broadcast_to(gl[:, 1:2], (_B, _K))
    anc_s = ancS.swapaxes(1, 2)
    anc_l = ancL.swapaxes(1, 2)

    oxs, oys, ows, oxl, oyl, owl = _sc_topk(
        qx_s, qy_s, anc_s, qx_l, qy_l, anc_l)

    w1t = W1.T
    w2t = W2.T
    b1r = b1.reshape(1, _D)
    b2r = b2.reshape(1, _D)

    def assemble(ox, oy, ow):
        cols = [ox.T.reshape(-1), oy.T.reshape(-1), ow.T.reshape(-1),
                jnp.zeros((_K * _B,), jnp.float32)]
        return jnp.stack(cols, axis=-1)

    hs = _tc_mlp(assemble(oxs, oys, ows), w1t, b1r, w2t, b2r)
    hl = _tc_mlp(assemble(oxl, oyl, owl), w1t, b1r, w2t, b2r)
    return (hs, hl)
